# baseline (speedup 1.0000x reference)
import functools

import jax
import jax.numpy as jnp
from jax import lax
from jax.experimental import pallas as pl
from jax.experimental.pallas import tpu as pltpu
from jax.experimental.pallas import tpu_sc as plsc

N_GRAPHS = 256
CHUNK = 256
G_PER_W = N_GRAPHS // 32
DC = 8


def _make_hist(n_nodes):
    mesh = plsc.VectorSubcoreMesh(core_axis_name="c", subcore_axis_name="s")
    per_w = (n_nodes + 31) // 32
    stage = ((per_w + 7) // 8 * 8) + 16

    @functools.partial(
        pl.kernel,
        mesh=mesh,
        compiler_params=pltpu.CompilerParams(needs_layout_passes=False),
        out_type=jax.ShapeDtypeStruct((32, N_GRAPHS), jnp.int32),
        scratch_types=[
            pltpu.VMEM((stage,), jnp.int32),
            pltpu.VMEM((N_GRAPHS,), jnp.int32),
        ],
    )
    def hist(batch_hbm, out_hbm, bv, cnt):
        w = lax.axis_index("s") * 2 + lax.axis_index("c")
        p0 = w * per_w
        p1 = jnp.minimum(p0 + per_w, n_nodes)
        a0 = pl.multiple_of((p0 // 8) * 8, 8)
        sh = pl.multiple_of(
            jnp.minimum(a0, ((n_nodes - stage) // 8) * 8), 8
        )
        pltpu.sync_copy(batch_hbm.at[pl.ds(sh, stage)], bv)
        zero = jnp.zeros((16,), jnp.int32)
        for c in range(N_GRAPHS // 16):
            cnt[pl.ds(c * 16, 16)] = zero
        ones = jnp.full((16,), 1, jnp.int32)
        lane = lax.iota(jnp.int32, 16)

        def body(j, _):
            v = bv[pl.ds(j * 16, 16)]
            p = sh + j * 16 + lane
            m = (p >= p0) & (p < p1)
            plsc.addupdate_scatter(cnt, [v], ones, mask=m)
            return 0

        lax.fori_loop(0, stage // 16, body, 0)
        pltpu.sync_copy(cnt, out_hbm.at[w])

    return hist


def _make_fused(n_nodes, dim):
    mesh = plsc.VectorSubcoreMesh(core_axis_name="c", subcore_axis_name="s")

    @functools.partial(
        pl.kernel,
        mesh=mesh,
        compiler_params=pltpu.CompilerParams(needs_layout_passes=False),
        out_type=jax.ShapeDtypeStruct((N_GRAPHS, dim), jnp.float32),
        scratch_types=[
            pltpu.VMEM((16,), jnp.int32),
            pltpu.VMEM((2, CHUNK, dim), jnp.float32),
            pltpu.VMEM((dim, dim), jnp.float32),
            pltpu.VMEM((G_PER_W, dim), jnp.float32),
            pltpu.VMEM((G_PER_W, dim), jnp.float32),
            pltpu.VMEM((DC, G_PER_W * 16), jnp.float32),
            pltpu.SemaphoreType.DMA,
            pltpu.SemaphoreType.DMA,
        ],
    )
    def fused(x_hbm, starts_hbm, w_hbm, out_hbm,
              sv, xbuf, wbuf, acc, tgq, mtq, sem0, sem1):
        w = lax.axis_index("s") * 2 + lax.axis_index("c")
        pltpu.sync_copy(starts_hbm.at[pl.ds(w * G_PER_W, 16)], sv)
        pltpu.sync_copy(w_hbm, wbuf)
        zero = jnp.zeros((16,), jnp.float32)
        for gi in range(G_PER_W):
            for c in range(DC):
                acc[gi, pl.ds(c * 16, 16)] = zero
        svv = sv[...]
        s_lo = svv[0]
        s_hi = svv[G_PER_W]
        base = (s_lo // 8) * 8
        nch = (s_hi - base + CHUNK - 1) // CHUNK

        def chunk_off(k):
            c0 = base + k * CHUNK
            return c0, pl.multiple_of(jnp.minimum(c0, n_nodes - CHUNK), 8)

        def start_dma(k, b, sem):
            _, off = chunk_off(k)
            pltpu.make_async_copy(
                x_hbm.at[pl.ds(off, CHUNK), :], b, sem
            ).start()

        def wait_dma(b, sem):
            pltpu.make_async_copy(
                x_hbm.at[pl.ds(0, CHUNK), :], b, sem
            ).wait()

        def run_pass(compute_rows):
            @pl.when(nch > 0)
            def _():
                start_dma(0, xbuf.at[0], sem0)

            def body(k, _):
                par = k % 2

                @pl.when(par == 0)
                def _():
                    wait_dma(xbuf.at[0], sem0)

                @pl.when(par == 1)
                def _():
                    wait_dma(xbuf.at[1], sem1)

                @pl.when(k + 1 < nch)
                def _():
                    @pl.when(par == 0)
                    def _():
                        start_dma(k + 1, xbuf.at[1], sem1)

                    @pl.when(par == 1)
                    def _():
                        start_dma(k + 1, xbuf.at[0], sem0)

                c0, off = chunk_off(k)
                c1 = jnp.minimum(c0 + CHUNK, s_hi)
                for gi in range(G_PER_W):
                    lo = jnp.maximum(svv[gi], c0)
                    hi = jnp.minimum(svv[gi + 1], c1)

                    @pl.when(hi > lo)
                    def _():
                        compute_rows(gi, lo, hi, off, par)
                return 0

            lax.fori_loop(0, nch, body, 0)

        def p1_rows(gi, lo, hi, off, par):
            init = tuple(acc[gi, pl.ds(c * 16, 16)] for c in range(DC))
            n = hi - lo
            half = n // 2

            def row2(i, carry):
                rl = lo - off + 2 * i
                return tuple(
                    carry[c]
                    + (xbuf[par, rl, pl.ds(c * 16, 16)]
                       + xbuf[par, rl + 1, pl.ds(c * 16, 16)])
                    for c in range(DC)
                )

            res = lax.fori_loop(0, half, row2, init)
            modd = (n % 2) == 1
            rt = hi - 1 - off
            res = tuple(
                res[c] + jnp.where(modd, xbuf[par, rt, pl.ds(c * 16, 16)], zero)
                for c in range(DC)
            )
            for c in range(DC):
                acc[gi, pl.ds(c * 16, 16)] = res[c]

        run_pass(p1_rows)

        for gi in range(G_PER_W):
            cnt = (svv[gi + 1] - svv[gi]).astype(jnp.float32)
            inv = 1.0 / jnp.maximum(jnp.full((16,), cnt, jnp.float32), 1.0)
            for c in range(DC):
                mtq[c, pl.ds(gi * 16, 16)] = acc[gi, pl.ds(c * 16, 16)] * inv

        for jb in range(2):
            def mm_body(kb, carry):
                mv = [mtq[kb, pl.ds(g * 16, 16)] for g in range(G_PER_W)]
                out = list(carry)
                for t in range(16):
                    wrow = [
                        wbuf[kb * 16 + t, pl.ds((jb * 4 + j) * 16, 16)]
                        for j in range(4)
                    ]
                    for g in range(G_PER_W):
                        s = mv[g][t]
                        for j in range(4):
                            out[g * 4 + j] = out[g * 4 + j] + s * wrow[j]
                return tuple(out)

            zeros32 = tuple(
                jnp.zeros((16,), jnp.float32) for _ in range(G_PER_W * 4)
            )
            res = lax.fori_loop(0, DC, mm_body, zeros32)
            res = list(res)
            for g in range(G_PER_W):
                for j in range(4):
                    a = res[g * 4 + j]
                    t = 1.0 - 2.0 / (jnp.exp(2.0 * a) + 1.0)
                    tgq[g, pl.ds((jb * 4 + j) * 16, 16)] = t

        for gi in range(G_PER_W):
            for c in range(DC):
                acc[gi, pl.ds(c * 16, 16)] = zero

        def p2_rows(gi, lo, hi, off, par):
            tgv = tuple(tgq[gi, pl.ds(c * 16, 16)] for c in range(DC))
            init = tuple(acc[gi, pl.ds(c * 16, 16)] for c in range(DC))

            def dot_coef(rl):
                p0 = (xbuf[par, rl, pl.ds(0, 16)] * tgv[0]
                      + xbuf[par, rl, pl.ds(16, 16)] * tgv[1])
                p1 = (xbuf[par, rl, pl.ds(32, 16)] * tgv[2]
                      + xbuf[par, rl, pl.ds(48, 16)] * tgv[3])
                p2 = (xbuf[par, rl, pl.ds(64, 16)] * tgv[4]
                      + xbuf[par, rl, pl.ds(80, 16)] * tgv[5])
                p3 = (xbuf[par, rl, pl.ds(96, 16)] * tgv[6]
                      + xbuf[par, rl, pl.ds(112, 16)] * tgv[7])
                part = (p0 + p1) + (p2 + p3)
                s = jnp.sum(part) * 10.0
                z = jnp.full((16,), s, jnp.float32)
                return 1.0 / (1.0 + jnp.exp(-z))

            n = hi - lo
            half = n // 2

            def row2(i, carry):
                rl = lo - off + 2 * i
                ca = dot_coef(rl)
                cb = dot_coef(rl + 1)
                return tuple(
                    carry[c]
                    + (ca * xbuf[par, rl, pl.ds(c * 16, 16)]
                       + cb * xbuf[par, rl + 1, pl.ds(c * 16, 16)])
                    for c in range(DC)
                )

            res = lax.fori_loop(0, half, row2, init)
            modd = (n % 2) == 1
            rt = hi - 1 - off
            ct = dot_coef(rt)
            res = tuple(
                res[c]
                + jnp.where(modd, ct * xbuf[par, rt, pl.ds(c * 16, 16)], zero)
                for c in range(DC)
            )
            for c in range(DC):
                acc[gi, pl.ds(c * 16, 16)] = res[c]

        run_pass(p2_rows)
        pltpu.sync_copy(acc, out_hbm.at[pl.ds(w * G_PER_W, G_PER_W), :])

    return fused


def kernel(x, batch, W):
    n_nodes, dim = x.shape
    batch = batch.astype(jnp.int32)
    parts = _make_hist(n_nodes)(batch)
    counts = parts.sum(axis=0)
    starts = jnp.concatenate(
        [jnp.zeros((1,), jnp.int32), jnp.cumsum(counts, dtype=jnp.int32)]
    )
    starts_ext = jnp.concatenate(
        [starts[:N_GRAPHS], jnp.full((16,), n_nodes, jnp.int32)]
    )
    return _make_fused(n_nodes, dim)(x, starts_ext, W)


# speedup vs baseline: 1.3193x; 1.3193x over previous
import functools

import jax
import jax.numpy as jnp
from jax import lax
from jax.experimental import pallas as pl
from jax.experimental.pallas import tpu as pltpu
from jax.experimental.pallas import tpu_sc as plsc

N_GRAPHS = 256
CHUNK = 256
G_PER_W = N_GRAPHS // 32
DC = 8


def _make_hist(n_nodes):
    mesh = plsc.VectorSubcoreMesh(core_axis_name="c", subcore_axis_name="s")
    per_w = (n_nodes + 31) // 32
    stage = ((per_w + 7) // 8 * 8) + 16

    @functools.partial(
        pl.kernel,
        mesh=mesh,
        compiler_params=pltpu.CompilerParams(needs_layout_passes=False),
        out_type=jax.ShapeDtypeStruct((32, N_GRAPHS), jnp.int32),
        scratch_types=[
            pltpu.VMEM((stage,), jnp.int32),
            pltpu.VMEM((N_GRAPHS,), jnp.int32),
        ],
    )
    def hist(batch_hbm, out_hbm, bv, cnt):
        w = lax.axis_index("s") * 2 + lax.axis_index("c")
        p0 = w * per_w
        p1 = jnp.minimum(p0 + per_w, n_nodes)
        a0 = pl.multiple_of((p0 // 8) * 8, 8)
        sh = pl.multiple_of(
            jnp.minimum(a0, ((n_nodes - stage) // 8) * 8), 8
        )
        pltpu.sync_copy(batch_hbm.at[pl.ds(sh, stage)], bv)
        zero = jnp.zeros((16,), jnp.int32)
        for c in range(N_GRAPHS // 16):
            cnt[pl.ds(c * 16, 16)] = zero
        ones = jnp.full((16,), 1, jnp.int32)
        lane = lax.iota(jnp.int32, 16)

        def body(j, _):
            v = bv[pl.ds(j * 16, 16)]
            p = sh + j * 16 + lane
            m = (p >= p0) & (p < p1)
            plsc.addupdate_scatter(cnt, [v], ones, mask=m)
            return 0

        lax.fori_loop(0, stage // 16, body, 0)
        pltpu.sync_copy(cnt, out_hbm.at[w])

    return hist


_GDN = lax.GatherDimensionNumbers(
    offset_dims=(), collapsed_slice_dims=(0,), start_index_map=(0,)
)


def _lane_bcast(v, idx16):
    return lax.gather(
        v, idx16.reshape(16, 1), _GDN, (1,),
        mode=lax.GatherScatterMode.PROMISE_IN_BOUNDS,
    )


def _make_fused(n_nodes, dim):
    mesh = plsc.VectorSubcoreMesh(core_axis_name="c", subcore_axis_name="s")

    @functools.partial(
        pl.kernel,
        mesh=mesh,
        compiler_params=pltpu.CompilerParams(needs_layout_passes=False),
        out_type=jax.ShapeDtypeStruct((N_GRAPHS, dim), jnp.float32),
        scratch_types=[
            pltpu.VMEM((16,), jnp.int32),
            pltpu.VMEM((2, CHUNK, dim), jnp.float32),
            pltpu.VMEM((dim, dim), jnp.float32),
            pltpu.VMEM((G_PER_W, dim), jnp.float32),
            pltpu.VMEM((G_PER_W, dim), jnp.float32),
            pltpu.VMEM((DC, G_PER_W * 16), jnp.float32),
            pltpu.SemaphoreType.DMA,
            pltpu.SemaphoreType.DMA,
        ],
    )
    def fused(x_hbm, starts_hbm, w_hbm, out_hbm,
              sv, xbuf, wbuf, acc, tgq, mtq, sem0, sem1):
        w = lax.axis_index("s") * 2 + lax.axis_index("c")
        pltpu.sync_copy(starts_hbm.at[pl.ds(w * G_PER_W, 16)], sv)
        pltpu.sync_copy(w_hbm, wbuf)
        zero = jnp.zeros((16,), jnp.float32)
        for gi in range(G_PER_W):
            for c in range(DC):
                acc[gi, pl.ds(c * 16, 16)] = zero
        svv = sv[...]
        s_lo = svv[0]
        s_hi = svv[G_PER_W]
        base = (s_lo // 8) * 8
        nch = (s_hi - base + CHUNK - 1) // CHUNK

        def chunk_off(k):
            c0 = base + k * CHUNK
            return c0, pl.multiple_of(jnp.minimum(c0, n_nodes - CHUNK), 8)

        def start_dma(k, b, sem):
            _, off = chunk_off(k)
            pltpu.make_async_copy(
                x_hbm.at[pl.ds(off, CHUNK), :], b, sem
            ).start()

        def wait_dma(b, sem):
            pltpu.make_async_copy(
                x_hbm.at[pl.ds(0, CHUNK), :], b, sem
            ).wait()

        def run_pass(compute_rows):
            @pl.when(nch > 0)
            def _():
                start_dma(0, xbuf.at[0], sem0)

            def body(k, _):
                par = k % 2

                @pl.when(par == 0)
                def _():
                    wait_dma(xbuf.at[0], sem0)

                @pl.when(par == 1)
                def _():
                    wait_dma(xbuf.at[1], sem1)

                @pl.when(k + 1 < nch)
                def _():
                    @pl.when(par == 0)
                    def _():
                        start_dma(k + 1, xbuf.at[1], sem1)

                    @pl.when(par == 1)
                    def _():
                        start_dma(k + 1, xbuf.at[0], sem0)

                c0, off = chunk_off(k)
                c1 = jnp.minimum(c0 + CHUNK, s_hi)
                for gi in range(G_PER_W):
                    lo = jnp.maximum(svv[gi], c0)
                    hi = jnp.minimum(svv[gi + 1], c1)

                    @pl.when(hi > lo)
                    def _():
                        compute_rows(gi, lo, hi, off, par)
                return 0

            lax.fori_loop(0, nch, body, 0)

        def p1_rows(gi, lo, hi, off, par):
            init = tuple(acc[gi, pl.ds(c * 16, 16)] for c in range(DC))
            n = hi - lo
            half = n // 2

            def row2(i, carry):
                rl = lo - off + 2 * i
                return tuple(
                    carry[c]
                    + (xbuf[par, rl, pl.ds(c * 16, 16)]
                       + xbuf[par, rl + 1, pl.ds(c * 16, 16)])
                    for c in range(DC)
                )

            res = lax.fori_loop(0, half, row2, init)
            modd = (n % 2) == 1
            rt = hi - 1 - off
            res = tuple(
                res[c] + jnp.where(modd, xbuf[par, rt, pl.ds(c * 16, 16)], zero)
                for c in range(DC)
            )
            for c in range(DC):
                acc[gi, pl.ds(c * 16, 16)] = res[c]

        run_pass(p1_rows)

        for gi in range(G_PER_W):
            cnt = (svv[gi + 1] - svv[gi]).astype(jnp.float32)
            inv = 1.0 / jnp.maximum(jnp.full((16,), cnt, jnp.float32), 1.0)
            for c in range(DC):
                mtq[c, pl.ds(gi * 16, 16)] = acc[gi, pl.ds(c * 16, 16)] * inv

        for jb in range(2):
            def mm_body(kb, carry):
                mv = [mtq[kb, pl.ds(g * 16, 16)] for g in range(G_PER_W)]
                out = list(carry)
                for t in range(16):
                    wrow = [
                        wbuf[kb * 16 + t, pl.ds((jb * 4 + j) * 16, 16)]
                        for j in range(4)
                    ]
                    tt = jnp.full((16,), t, jnp.int32)
                    for g in range(G_PER_W):
                        s = _lane_bcast(mv[g], tt)
                        for j in range(4):
                            out[g * 4 + j] = out[g * 4 + j] + s * wrow[j]
                return tuple(out)

            zeros32 = tuple(
                jnp.zeros((16,), jnp.float32) for _ in range(G_PER_W * 4)
            )
            res = lax.fori_loop(0, DC, mm_body, zeros32)
            res = list(res)
            for g in range(G_PER_W):
                for j in range(4):
                    a = res[g * 4 + j]
                    t = 1.0 - 2.0 / (jnp.exp(2.0 * a) + 1.0)
                    tgq[g, pl.ds((jb * 4 + j) * 16, 16)] = t

        for gi in range(G_PER_W):
            for c in range(DC):
                acc[gi, pl.ds(c * 16, 16)] = zero

        def p2_rows(gi, lo, hi, off, par):
            tgv = tuple(tgq[gi, pl.ds(c * 16, 16)] for c in range(DC))
            init = tuple(acc[gi, pl.ds(c * 16, 16)] for c in range(DC))

            def row(r, carry):
                rl = r - off
                xv = [xbuf[par, rl, pl.ds(c * 16, 16)] for c in range(DC)]
                p0 = xv[0] * tgv[0] + xv[1] * tgv[1]
                p1 = xv[2] * tgv[2] + xv[3] * tgv[3]
                p2 = xv[4] * tgv[4] + xv[5] * tgv[5]
                p3 = xv[6] * tgv[6] + xv[7] * tgv[7]
                part = (p0 + p1) + (p2 + p3)
                s = jnp.sum(part) * 10.0
                z = jnp.full((16,), s, jnp.float32)
                coef = 1.0 / (1.0 + jnp.exp(-z))
                return tuple(carry[c] + coef * xv[c] for c in range(DC))

            res = lax.fori_loop(lo, hi, row, init)
            for c in range(DC):
                acc[gi, pl.ds(c * 16, 16)] = res[c]

        run_pass(p2_rows)
        pltpu.sync_copy(acc, out_hbm.at[pl.ds(w * G_PER_W, G_PER_W), :])

    return fused


def kernel(x, batch, W):
    n_nodes, dim = x.shape
    batch = batch.astype(jnp.int32)
    parts = _make_hist(n_nodes)(batch)
    counts = parts.sum(axis=0)
    starts = jnp.concatenate(
        [jnp.zeros((1,), jnp.int32), jnp.cumsum(counts, dtype=jnp.int32)]
    )
    starts_ext = jnp.concatenate(
        [starts[:N_GRAPHS], jnp.full((16,), n_nodes, jnp.int32)]
    )
    return _make_fused(n_nodes, dim)(x, starts_ext, W)


# speedup vs baseline: 1.4243x; 1.0796x over previous
import functools

import jax
import jax.numpy as jnp
from jax import lax
from jax.experimental import pallas as pl
from jax.experimental.pallas import tpu as pltpu
from jax.experimental.pallas import tpu_sc as plsc

N_GRAPHS = 256
CHUNK = 256
G_PER_W = N_GRAPHS // 32
DC = 8


def _make_hist(n_nodes):
    mesh = plsc.VectorSubcoreMesh(core_axis_name="c", subcore_axis_name="s")
    per_w = (n_nodes + 31) // 32
    stage = ((per_w + 7) // 8 * 8) + 16

    @functools.partial(
        pl.kernel,
        mesh=mesh,
        compiler_params=pltpu.CompilerParams(needs_layout_passes=False),
        out_type=jax.ShapeDtypeStruct((32, N_GRAPHS), jnp.int32),
        scratch_types=[
            pltpu.VMEM((stage,), jnp.int32),
            pltpu.VMEM((N_GRAPHS,), jnp.int32),
        ],
    )
    def hist(batch_hbm, out_hbm, bv, cnt):
        w = lax.axis_index("s") * 2 + lax.axis_index("c")
        p0 = w * per_w
        p1 = jnp.minimum(p0 + per_w, n_nodes)
        a0 = pl.multiple_of((p0 // 8) * 8, 8)
        sh = pl.multiple_of(
            jnp.minimum(a0, ((n_nodes - stage) // 8) * 8), 8
        )
        pltpu.sync_copy(batch_hbm.at[pl.ds(sh, stage)], bv)
        zero = jnp.zeros((16,), jnp.int32)
        for c in range(N_GRAPHS // 16):
            cnt[pl.ds(c * 16, 16)] = zero
        ones = jnp.full((16,), 1, jnp.int32)
        lane = lax.iota(jnp.int32, 16)

        def body(j, _):
            v = bv[pl.ds(j * 16, 16)]
            p = sh + j * 16 + lane
            m = (p >= p0) & (p < p1)
            plsc.addupdate_scatter(cnt, [v], ones, mask=m)
            return 0

        lax.fori_loop(0, stage // 16, body, 0)
        pltpu.sync_copy(cnt, out_hbm.at[w])

    return hist


def _make_fused(n_nodes, dim):
    mesh = plsc.VectorSubcoreMesh(core_axis_name="c", subcore_axis_name="s")

    @functools.partial(
        pl.kernel,
        mesh=mesh,
        compiler_params=pltpu.CompilerParams(needs_layout_passes=False),
        out_type=jax.ShapeDtypeStruct((N_GRAPHS, dim), jnp.float32),
        scratch_types=[
            pltpu.VMEM((2, CHUNK, dim), jnp.float32),
            pltpu.VMEM((dim, dim), jnp.float32),
            pltpu.VMEM((G_PER_W, dim), jnp.float32),
            pltpu.VMEM((G_PER_W, dim), jnp.float32),
            pltpu.VMEM((DC, G_PER_W * 16), jnp.float32),
            pltpu.VMEM((32, N_GRAPHS), jnp.int32),
            pltpu.VMEM((N_GRAPHS + 16,), jnp.int32),
            pltpu.SemaphoreType.DMA,
            pltpu.SemaphoreType.DMA,
        ],
    )
    def fused(x_hbm, parts_hbm, w_hbm, out_hbm,
              xbuf, wbuf, acc, tgq, mtq, pbuf, svf, sem0, sem1):
        w = lax.axis_index("s") * 2 + lax.axis_index("c")
        pltpu.sync_copy(parts_hbm, pbuf)
        pltpu.sync_copy(w_hbm, wbuf)
        run = jnp.int32(0)
        for c in range(N_GRAPHS // 16):
            s_c = pbuf[0, pl.ds(c * 16, 16)]
            for r in range(1, 32):
                s_c = s_c + pbuf[r, pl.ds(c * 16, 16)]
            incl = jnp.cumsum(s_c)
            svf[pl.ds(c * 16, 16)] = (incl - s_c) + run
            run = run + incl[15]
        svf[pl.ds(N_GRAPHS, 16)] = jnp.full((16,), n_nodes, jnp.int32)
        zero = jnp.zeros((16,), jnp.float32)
        for gi in range(G_PER_W):
            for c in range(DC):
                acc[gi, pl.ds(c * 16, 16)] = zero
        svv = svf[pl.ds(pl.multiple_of(w * G_PER_W, 8), 16)]
        s_lo = svv[0]
        s_hi = svv[G_PER_W]
        base = (s_lo // 8) * 8
        nch = (s_hi - base + CHUNK - 1) // CHUNK

        def chunk_off(k):
            c0 = base + k * CHUNK
            return c0, pl.multiple_of(jnp.minimum(c0, n_nodes - CHUNK), 8)

        def start_dma(k, b, sem):
            _, off = chunk_off(k)
            pltpu.make_async_copy(
                x_hbm.at[pl.ds(off, CHUNK), :], b, sem
            ).start()

        def wait_dma(b, sem):
            pltpu.make_async_copy(
                x_hbm.at[pl.ds(0, CHUNK), :], b, sem
            ).wait()

        def run_pass(compute_rows):
            @pl.when(nch > 0)
            def _():
                start_dma(0, xbuf.at[0], sem0)

            def body(k, _):
                par = k % 2

                @pl.when(par == 0)
                def _():
                    wait_dma(xbuf.at[0], sem0)

                @pl.when(par == 1)
                def _():
                    wait_dma(xbuf.at[1], sem1)

                @pl.when(k + 1 < nch)
                def _():
                    @pl.when(par == 0)
                    def _():
                        start_dma(k + 1, xbuf.at[1], sem1)

                    @pl.when(par == 1)
                    def _():
                        start_dma(k + 1, xbuf.at[0], sem0)

                c0, off = chunk_off(k)
                c1 = jnp.minimum(c0 + CHUNK, s_hi)
                for gi in range(G_PER_W):
                    lo = jnp.maximum(svv[gi], c0)
                    hi = jnp.minimum(svv[gi + 1], c1)

                    @pl.when(hi > lo)
                    def _():
                        compute_rows(gi, lo, hi, off, par)
                return 0

            lax.fori_loop(0, nch, body, 0)

        def p1_rows(gi, lo, hi, off, par):
            init = tuple(acc[gi, pl.ds(c * 16, 16)] for c in range(DC))
            n = hi - lo
            half = n // 2

            def row2(i, carry):
                rl = lo - off + 2 * i
                return tuple(
                    carry[c]
                    + (xbuf[par, rl, pl.ds(c * 16, 16)]
                       + xbuf[par, rl + 1, pl.ds(c * 16, 16)])
                    for c in range(DC)
                )

            res = lax.fori_loop(0, half, row2, init)
            modd = (n % 2) == 1
            rt = hi - 1 - off
            res = tuple(
                res[c] + jnp.where(modd, xbuf[par, rt, pl.ds(c * 16, 16)], zero)
                for c in range(DC)
            )
            for c in range(DC):
                acc[gi, pl.ds(c * 16, 16)] = res[c]

        run_pass(p1_rows)

        for gi in range(G_PER_W):
            cnt = (svv[gi + 1] - svv[gi]).astype(jnp.float32)
            inv = 1.0 / jnp.maximum(jnp.full((16,), cnt, jnp.float32), 1.0)
            for c in range(DC):
                mtq[c, pl.ds(gi * 16, 16)] = acc[gi, pl.ds(c * 16, 16)] * inv

        for jb in range(2):
            def mm_body(kb, carry):
                mv = [mtq[kb, pl.ds(g * 16, 16)] for g in range(G_PER_W)]
                out = list(carry)
                for t in range(16):
                    wrow = [
                        wbuf[kb * 16 + t, pl.ds((jb * 4 + j) * 16, 16)]
                        for j in range(4)
                    ]
                    for g in range(G_PER_W):
                        s = mv[g][t]
                        for j in range(4):
                            out[g * 4 + j] = out[g * 4 + j] + s * wrow[j]
                return tuple(out)

            zeros32 = tuple(
                jnp.zeros((16,), jnp.float32) for _ in range(G_PER_W * 4)
            )
            res = lax.fori_loop(0, DC, mm_body, zeros32)
            res = list(res)
            for g in range(G_PER_W):
                for j in range(4):
                    a = res[g * 4 + j]
                    t = 1.0 - 2.0 / (jnp.exp(2.0 * a) + 1.0)
                    tgq[g, pl.ds((jb * 4 + j) * 16, 16)] = t

        for gi in range(G_PER_W):
            for c in range(DC):
                acc[gi, pl.ds(c * 16, 16)] = zero

        def p2_rows(gi, lo, hi, off, par):
            tgv = tuple(tgq[gi, pl.ds(c * 16, 16)] for c in range(DC))
            init = tuple(acc[gi, pl.ds(c * 16, 16)] for c in range(DC))

            def row(r, carry):
                rl = r - off
                xv = [xbuf[par, rl, pl.ds(c * 16, 16)] for c in range(DC)]
                p0 = xv[0] * tgv[0] + xv[1] * tgv[1]
                p1 = xv[2] * tgv[2] + xv[3] * tgv[3]
                p2 = xv[4] * tgv[4] + xv[5] * tgv[5]
                p3 = xv[6] * tgv[6] + xv[7] * tgv[7]
                part = (p0 + p1) + (p2 + p3)
                s = jnp.sum(part) * 10.0
                z = jnp.full((16,), s, jnp.float32)
                coef = 1.0 / (1.0 + jnp.exp(-z))
                return tuple(carry[c] + coef * xv[c] for c in range(DC))

            res = lax.fori_loop(lo, hi, row, init)
            for c in range(DC):
                acc[gi, pl.ds(c * 16, 16)] = res[c]

        run_pass(p2_rows)
        pltpu.sync_copy(acc, out_hbm.at[pl.ds(w * G_PER_W, G_PER_W), :])

    return fused


def kernel(x, batch, W):
    n_nodes, dim = x.shape
    batch = batch.astype(jnp.int32)
    parts = _make_hist(n_nodes)(batch)
    return _make_fused(n_nodes, dim)(x, parts, W)


# speedup vs baseline: 1.4833x; 1.0414x over previous
import functools

import jax
import jax.numpy as jnp
from jax import lax
from jax.experimental import pallas as pl
from jax.experimental.pallas import tpu as pltpu
from jax.experimental.pallas import tpu_sc as plsc

N_GRAPHS = 256
CHUNK = 384
G_PER_W = N_GRAPHS // 32
DC = 8


def _make_hist(n_nodes):
    mesh = plsc.VectorSubcoreMesh(core_axis_name="c", subcore_axis_name="s")
    per_w = (n_nodes + 31) // 32
    stage = ((per_w + 7) // 8 * 8) + 16

    @functools.partial(
        pl.kernel,
        mesh=mesh,
        compiler_params=pltpu.CompilerParams(needs_layout_passes=False),
        out_type=jax.ShapeDtypeStruct((32, N_GRAPHS), jnp.int32),
        scratch_types=[
            pltpu.VMEM((stage,), jnp.int32),
            pltpu.VMEM((N_GRAPHS,), jnp.int32),
        ],
    )
    def hist(batch_hbm, out_hbm, bv, cnt):
        w = lax.axis_index("s") * 2 + lax.axis_index("c")
        p0 = w * per_w
        p1 = jnp.minimum(p0 + per_w, n_nodes)
        a0 = pl.multiple_of((p0 // 8) * 8, 8)
        sh = pl.multiple_of(
            jnp.minimum(a0, ((n_nodes - stage) // 8) * 8), 8
        )
        pltpu.sync_copy(batch_hbm.at[pl.ds(sh, stage)], bv)
        zero = jnp.zeros((16,), jnp.int32)
        for c in range(N_GRAPHS // 16):
            cnt[pl.ds(c * 16, 16)] = zero
        ones = jnp.full((16,), 1, jnp.int32)
        lane = lax.iota(jnp.int32, 16)

        def body(j, _):
            v = bv[pl.ds(j * 16, 16)]
            p = sh + j * 16 + lane
            m = (p >= p0) & (p < p1)
            plsc.addupdate_scatter(cnt, [v], ones, mask=m)
            return 0

        lax.fori_loop(0, stage // 16, body, 0)
        pltpu.sync_copy(cnt, out_hbm.at[w])

    return hist


def _make_fused(n_nodes, dim):
    mesh = plsc.VectorSubcoreMesh(core_axis_name="c", subcore_axis_name="s")

    @functools.partial(
        pl.kernel,
        mesh=mesh,
        compiler_params=pltpu.CompilerParams(needs_layout_passes=False),
        out_type=jax.ShapeDtypeStruct((N_GRAPHS, dim), jnp.float32),
        scratch_types=[
            pltpu.VMEM((16,), jnp.int32),
            pltpu.VMEM((2, CHUNK, dim), jnp.float32),
            pltpu.VMEM((dim, dim), jnp.float32),
            pltpu.VMEM((G_PER_W, dim), jnp.float32),
            pltpu.VMEM((G_PER_W, dim), jnp.float32),
            pltpu.VMEM((DC, G_PER_W * 16), jnp.float32),
            pltpu.SemaphoreType.DMA,
            pltpu.SemaphoreType.DMA,
        ],
    )
    def fused(x_hbm, starts_hbm, w_hbm, out_hbm,
              sv, xbuf, wbuf, acc, tgq, mtq, sem0, sem1):
        w = lax.axis_index("s") * 2 + lax.axis_index("c")
        pltpu.sync_copy(starts_hbm.at[pl.ds(w * G_PER_W, 16)], sv)
        pltpu.sync_copy(w_hbm, wbuf)
        zero = jnp.zeros((16,), jnp.float32)
        for gi in range(G_PER_W):
            for c in range(DC):
                acc[gi, pl.ds(c * 16, 16)] = zero
        svv = sv[...]
        s_lo = svv[0]
        s_hi = svv[G_PER_W]
        base = (s_lo // 8) * 8
        nch = (s_hi - base + CHUNK - 1) // CHUNK

        def chunk_off(k):
            c0 = base + k * CHUNK
            return c0, pl.multiple_of(jnp.minimum(c0, n_nodes - CHUNK), 8)

        def start_dma(k, b, sem):
            _, off = chunk_off(k)
            pltpu.make_async_copy(
                x_hbm.at[pl.ds(off, CHUNK), :], b, sem
            ).start()

        def wait_dma(b, sem):
            pltpu.make_async_copy(
                x_hbm.at[pl.ds(0, CHUNK), :], b, sem
            ).wait()

        def run_pass(compute_rows):
            @pl.when(nch > 0)
            def _():
                start_dma(0, xbuf.at[0], sem0)

            def body(k, _):
                par = k % 2

                @pl.when(par == 0)
                def _():
                    wait_dma(xbuf.at[0], sem0)

                @pl.when(par == 1)
                def _():
                    wait_dma(xbuf.at[1], sem1)

                @pl.when(k + 1 < nch)
                def _():
                    @pl.when(par == 0)
                    def _():
                        start_dma(k + 1, xbuf.at[1], sem1)

                    @pl.when(par == 1)
                    def _():
                        start_dma(k + 1, xbuf.at[0], sem0)

                c0, off = chunk_off(k)
                c1 = jnp.minimum(c0 + CHUNK, s_hi)
                for gi in range(G_PER_W):
                    lo = jnp.maximum(svv[gi], c0)
                    hi = jnp.minimum(svv[gi + 1], c1)

                    @pl.when(hi > lo)
                    def _():
                        compute_rows(gi, lo, hi, off, par)
                return 0

            lax.fori_loop(0, nch, body, 0)

        def p1_rows(gi, lo, hi, off, par):
            init = tuple(acc[gi, pl.ds(c * 16, 16)] for c in range(DC))
            n = hi - lo
            half = n // 2

            def row2(i, carry):
                rl = lo - off + 2 * i
                return tuple(
                    carry[c]
                    + (xbuf[par, rl, pl.ds(c * 16, 16)]
                       + xbuf[par, rl + 1, pl.ds(c * 16, 16)])
                    for c in range(DC)
                )

            res = lax.fori_loop(0, half, row2, init)
            modd = (n % 2) == 1
            rt = hi - 1 - off
            res = tuple(
                res[c] + jnp.where(modd, xbuf[par, rt, pl.ds(c * 16, 16)], zero)
                for c in range(DC)
            )
            for c in range(DC):
                acc[gi, pl.ds(c * 16, 16)] = res[c]

        run_pass(p1_rows)

        for gi in range(G_PER_W):
            cnt = (svv[gi + 1] - svv[gi]).astype(jnp.float32)
            inv = 1.0 / jnp.maximum(jnp.full((16,), cnt, jnp.float32), 1.0)
            for c in range(DC):
                mtq[c, pl.ds(gi * 16, 16)] = acc[gi, pl.ds(c * 16, 16)] * inv

        for jb in range(2):
            def mm_body(kb, carry):
                mv = [mtq[kb, pl.ds(g * 16, 16)] for g in range(G_PER_W)]
                out = list(carry)
                for t in range(16):
                    wrow = [
                        wbuf[kb * 16 + t, pl.ds((jb * 4 + j) * 16, 16)]
                        for j in range(4)
                    ]
                    for g in range(G_PER_W):
                        s = mv[g][t]
                        for j in range(4):
                            out[g * 4 + j] = out[g * 4 + j] + s * wrow[j]
                return tuple(out)

            zeros32 = tuple(
                jnp.zeros((16,), jnp.float32) for _ in range(G_PER_W * 4)
            )
            res = lax.fori_loop(0, DC, mm_body, zeros32)
            res = list(res)
            for g in range(G_PER_W):
                for j in range(4):
                    a = res[g * 4 + j]
                    t = 1.0 - 2.0 / (jnp.exp(2.0 * a) + 1.0)
                    tgq[g, pl.ds((jb * 4 + j) * 16, 16)] = t

        for gi in range(G_PER_W):
            for c in range(DC):
                acc[gi, pl.ds(c * 16, 16)] = zero

        def p2_rows(gi, lo, hi, off, par):
            tgv = tuple(tgq[gi, pl.ds(c * 16, 16)] for c in range(DC))
            init = tuple(acc[gi, pl.ds(c * 16, 16)] for c in range(DC))

            def row(r, carry):
                rl = r - off
                xv = [xbuf[par, rl, pl.ds(c * 16, 16)] for c in range(DC)]
                p0 = xv[0] * tgv[0] + xv[1] * tgv[1]
                p1 = xv[2] * tgv[2] + xv[3] * tgv[3]
                p2 = xv[4] * tgv[4] + xv[5] * tgv[5]
                p3 = xv[6] * tgv[6] + xv[7] * tgv[7]
                part = (p0 + p1) + (p2 + p3)
                s = jnp.sum(part) * 10.0
                z = jnp.full((16,), s, jnp.float32)
                coef = 1.0 / (1.0 + jnp.exp(-z))
                return tuple(carry[c] + coef * xv[c] for c in range(DC))

            res = lax.fori_loop(lo, hi, row, init)
            for c in range(DC):
                acc[gi, pl.ds(c * 16, 16)] = res[c]

        run_pass(p2_rows)
        pltpu.sync_copy(acc, out_hbm.at[pl.ds(w * G_PER_W, G_PER_W), :])

    return fused


def kernel(x, batch, W):
    n_nodes, dim = x.shape
    batch = batch.astype(jnp.int32)
    parts = _make_hist(n_nodes)(batch)
    counts = parts.sum(axis=0)
    starts = jnp.concatenate(
        [jnp.zeros((1,), jnp.int32), jnp.cumsum(counts, dtype=jnp.int32)]
    )
    starts_ext = jnp.concatenate(
        [starts[:N_GRAPHS], jnp.full((16,), n_nodes, jnp.int32)]
    )
    return _make_fused(n_nodes, dim)(x, starts_ext, W)
